# Initial kernel scaffold; baseline (speedup 1.0000x reference)
import jax
import jax.numpy as jnp
from jax.experimental import pallas as pl


def kernel(q, k, v, Wq, bq, Wk, bk, Wv, bv, Wo, bo):
    raise NotImplementedError("write your pallas kernel here")


# speedup vs baseline: 5.0370x; 5.0370x over previous
import functools
import math

import jax
import jax.numpy as jnp
from jax.experimental import pallas as pl

B, L, D, H = 4, 2048, 1024, 16
DK = D // H
F = L // 2

_HI = jax.lax.Precision.HIGHEST


def _dft_matrices():
    t = jnp.arange(L, dtype=jnp.int32)
    f = jnp.arange(F, dtype=jnp.int32)
    scale = jnp.float32(2.0 * math.pi / L)
    ang_f = ((f[:, None] * t[None, :]) % L).astype(jnp.float32) * scale
    fcos = jnp.cos(ang_f)
    fsin = jnp.sin(ang_f)
    alt_t = (1 - 2 * (t % 2)).astype(jnp.float32)
    fsin = fsin.at[0].set(alt_t)
    fmat = jnp.concatenate([fcos, fsin], axis=0)
    tau = jnp.arange(L, dtype=jnp.int32)
    ang_i = ((tau[:, None] * f[None, :]) % L).astype(jnp.float32) * scale
    icos = (2.0 / L) * jnp.cos(ang_i)
    icos = icos.at[:, 0].set(1.0 / L)
    isin = (-2.0 / L) * jnp.sin(ang_i)
    alt_tau = (1 - 2 * (tau % 2)).astype(jnp.float32) / L
    isin = isin.at[:, 0].set(alt_tau)
    icat = jnp.concatenate([icos, isin], axis=1)
    return fmat, icat


def _proj_kernel(q_ref, k_ref, v_ref, wq_ref, wk_ref, wv_ref,
                 bq_ref, bk_ref, bv_ref, qp_ref, kp_ref, vp_ref):
    qp_ref[...] = jnp.dot(q_ref[...].astype(jnp.bfloat16),
                          wq_ref[...].astype(jnp.bfloat16),
                          preferred_element_type=jnp.float32) + bq_ref[...]
    kp_ref[...] = jnp.dot(k_ref[...].astype(jnp.bfloat16),
                          wk_ref[...].astype(jnp.bfloat16),
                          preferred_element_type=jnp.float32) + bk_ref[...]
    vp_ref[...] = jnp.dot(v_ref[...].astype(jnp.bfloat16),
                          wv_ref[...].astype(jnp.bfloat16),
                          preferred_element_type=jnp.float32) + bv_ref[...]


def _projections(q2, k2, v2, wqt, wkt, wvt, bq2, bk2, bv2):
    m = B * L
    bm = 512
    row = pl.BlockSpec((bm, D), lambda i: (i, 0))
    full = pl.BlockSpec((D, D), lambda i: (0, 0))
    bias = pl.BlockSpec((1, D), lambda i: (0, 0))
    out = jax.ShapeDtypeStruct((m, D), jnp.float32)
    return pl.pallas_call(
        _proj_kernel,
        grid=(m // bm,),
        in_specs=[row, row, row, full, full, full, bias, bias, bias],
        out_specs=[row, row, row],
        out_shape=[out, out, out],
    )(q2, k2, v2, wqt, wkt, wvt, bq2, bk2, bv2)


def _fwd_kernel(f_ref, qp_ref, kp_ref, fq_ref, fk_ref):
    fq_ref[0] = jnp.dot(f_ref[...], qp_ref[0], precision=_HI)
    fk_ref[0] = jnp.dot(f_ref[...], kp_ref[0], precision=_HI)


def _forward_dft(fmat, qp3, kp3):
    bf, bc = 512, 512
    out = jax.ShapeDtypeStruct((B, L, D), jnp.float32)
    return pl.pallas_call(
        _fwd_kernel,
        grid=(B, L // bf, D // bc),
        in_specs=[
            pl.BlockSpec((bf, L), lambda b, i, j: (i, 0)),
            pl.BlockSpec((1, L, bc), lambda b, i, j: (b, 0, j)),
            pl.BlockSpec((1, L, bc), lambda b, i, j: (b, 0, j)),
        ],
        out_specs=[
            pl.BlockSpec((1, bf, bc), lambda b, i, j: (b, i, j)),
            pl.BlockSpec((1, bf, bc), lambda b, i, j: (b, i, j)),
        ],
        out_shape=[out, out],
    )(fmat, qp3, kp3)


def _point_kernel(fq_ref, fk_ref, p_ref):
    cq = fq_ref[0, :F, :]
    sq = fq_ref[0, F:, :]
    ck = fk_ref[0, :F, :]
    sk = fk_ref[0, F:, :]
    row0 = jax.lax.broadcasted_iota(jnp.int32, cq.shape, 0) == 0
    a = cq * ck + jnp.where(row0, 0.0, sq * sk)
    b = jnp.where(row0, sq * sk, cq * sk - sq * ck)
    p_ref[0, :F, :] = a
    p_ref[0, F:, :] = b


def _pointwise(fq, fk):
    bc = 512
    return pl.pallas_call(
        _point_kernel,
        grid=(B, D // bc),
        in_specs=[
            pl.BlockSpec((1, L, bc), lambda b, j: (b, 0, j)),
            pl.BlockSpec((1, L, bc), lambda b, j: (b, 0, j)),
        ],
        out_specs=pl.BlockSpec((1, L, bc), lambda b, j: (b, 0, j)),
        out_shape=jax.ShapeDtypeStruct((B, L, D), jnp.float32),
    )(fq, fk)


NT = 16


def _inv_kernel(i_ref, p_ref, mx_ref):
    ti = pl.program_id(2)
    corr = jnp.dot(i_ref[...], p_ref[0], precision=_HI)

    @pl.when(ti == 0)
    def _():
        row0 = jax.lax.broadcasted_iota(jnp.int32, corr.shape, 0) == 0
        c = jnp.where(row0, -jnp.inf, corr)
        mx_ref[0, 0] = jnp.broadcast_to(jnp.max(c, axis=0, keepdims=True),
                                        mx_ref.shape[2:])

    @pl.when(ti != 0)
    def _():
        m = jnp.max(corr, axis=0, keepdims=True)
        mx_ref[0, 0] = jnp.broadcast_to(m, mx_ref.shape[2:])


def _inv_max(icat, pcat):
    bt, bc = L // NT, 512
    return pl.pallas_call(
        _inv_kernel,
        grid=(B, D // bc, NT),
        in_specs=[
            pl.BlockSpec((bt, L), lambda b, j, i: (i, 0)),
            pl.BlockSpec((1, L, bc), lambda b, j, i: (b, 0, j)),
        ],
        out_specs=pl.BlockSpec((1, 1, 8, bc), lambda b, j, i: (b, i, 0, j)),
        out_shape=jax.ShapeDtypeStruct((B, NT, 8, D), jnp.float32),
    )(icat, pcat)


def _shifts(tilemax):
    c2s = tilemax[:, :, 0, :].reshape(B, NT, H, DK).transpose(0, 2, 3, 1)
    delays, _ = jax.lax.top_k(c2s, 7)
    delays = jnp.moveaxis(delays, -1, 1)
    sm = jax.nn.softmax(delays, axis=-1)
    return sm[:, 0].astype(jnp.int32).astype(jnp.float32).reshape(B, 1, D)


def _out_kernel(sh_ref, vp_ref, vprev_ref, wo_ref, bo_ref, o_ref):
    sh = sh_ref[0]
    vt = vp_ref[0]
    prev_last = vprev_ref[0, -1:, :]
    rolled = jnp.concatenate([prev_last, vt[:-1, :]], axis=0)
    sel = jnp.where(sh > 0.5, rolled, vt)
    o_ref[0] = jnp.dot(sel.astype(jnp.bfloat16),
                       wo_ref[...].astype(jnp.bfloat16),
                       preferred_element_type=jnp.float32) + bo_ref[...]


def _final(shifts, vp3, wot, bo2):
    bm = 512
    nm = L // bm
    return pl.pallas_call(
        _out_kernel,
        grid=(B, nm),
        in_specs=[
            pl.BlockSpec((1, 1, D), lambda b, i: (b, 0, 0)),
            pl.BlockSpec((1, bm, D), lambda b, i: (b, i, 0)),
            pl.BlockSpec((1, bm, D), lambda b, i: (b, (i - 1) % nm, 0)),
            pl.BlockSpec((D, D), lambda b, i: (0, 0)),
            pl.BlockSpec((1, D), lambda b, i: (0, 0)),
        ],
        out_specs=pl.BlockSpec((1, bm, D), lambda b, i: (b, i, 0)),
        out_shape=jax.ShapeDtypeStruct((B, L, D), jnp.float32),
    )(shifts, vp3, vp3, wot, bo2)


def kernel(q, k, v, Wq, bq, Wk, bk, Wv, bv, Wo, bo):
    fmat, icat = _dft_matrices()
    q2 = q.reshape(B * L, D)
    k2 = k.reshape(B * L, D)
    v2 = v.reshape(B * L, D)
    qp, kp, vp = _projections(q2, k2, v2, Wq.T, Wk.T, Wv.T,
                              bq.reshape(1, D), bk.reshape(1, D),
                              bv.reshape(1, D))
    qp3 = qp.reshape(B, L, D)
    kp3 = kp.reshape(B, L, D)
    vp3 = vp.reshape(B, L, D)
    fq, fk = _forward_dft(fmat, qp3, kp3)
    pcat = _pointwise(fq, fk)
    tilemax = _inv_max(icat, pcat)
    sh = _shifts(tilemax)
    out = _final(sh, vp3, Wo.T, bo.reshape(1, D))
    return out


# speedup vs baseline: 5.4404x; 1.0801x over previous
import functools
import math

import jax
import jax.numpy as jnp
from jax.experimental import pallas as pl

B, L, D, H = 4, 2048, 1024, 16
DK = D // H
F = L // 2

_HI = jax.lax.Precision.HIGHEST


def _dft_matrices():
    t = jnp.arange(L, dtype=jnp.int32)
    f = jnp.arange(F, dtype=jnp.int32)
    scale = jnp.float32(2.0 * math.pi / L)
    ang_f = ((f[:, None] * t[None, :]) % L).astype(jnp.float32) * scale
    fcos = jnp.cos(ang_f)
    fsin = jnp.sin(ang_f)
    alt_t = (1 - 2 * (t % 2)).astype(jnp.float32)
    fsin = fsin.at[0].set(alt_t)
    fmat = jnp.concatenate([fcos, fsin], axis=0)
    tau = jnp.arange(L, dtype=jnp.int32)
    ang_i = ((tau[:, None] * f[None, :]) % L).astype(jnp.float32) * scale
    icos = (2.0 / L) * jnp.cos(ang_i)
    icos = icos.at[:, 0].set(1.0 / L)
    isin = (-2.0 / L) * jnp.sin(ang_i)
    alt_tau = (1 - 2 * (tau % 2)).astype(jnp.float32) / L
    isin = isin.at[:, 0].set(alt_tau)
    icat = jnp.concatenate([icos, isin], axis=1)
    return fmat, icat


def _proj_kernel(q_ref, k_ref, v_ref, wq_ref, wk_ref, wv_ref,
                 bq_ref, bk_ref, bv_ref, qp_ref, kp_ref, vp_ref):
    qp_ref[...] = jnp.dot(q_ref[...].astype(jnp.bfloat16),
                          wq_ref[...].astype(jnp.bfloat16),
                          preferred_element_type=jnp.float32) + bq_ref[...]
    kp_ref[...] = jnp.dot(k_ref[...].astype(jnp.bfloat16),
                          wk_ref[...].astype(jnp.bfloat16),
                          preferred_element_type=jnp.float32) + bk_ref[...]
    vp_ref[...] = jnp.dot(v_ref[...].astype(jnp.bfloat16),
                          wv_ref[...].astype(jnp.bfloat16),
                          preferred_element_type=jnp.float32) + bv_ref[...]


def _projections(q2, k2, v2, wqt, wkt, wvt, bq2, bk2, bv2):
    m = B * L
    bm = 512
    row = pl.BlockSpec((bm, D), lambda i: (i, 0))
    full = pl.BlockSpec((D, D), lambda i: (0, 0))
    bias = pl.BlockSpec((1, D), lambda i: (0, 0))
    out = jax.ShapeDtypeStruct((m, D), jnp.float32)
    return pl.pallas_call(
        _proj_kernel,
        grid=(m // bm,),
        in_specs=[row, row, row, full, full, full, bias, bias, bias],
        out_specs=[row, row, row],
        out_shape=[out, out, out],
    )(q2, k2, v2, wqt, wkt, wvt, bq2, bk2, bv2)


def _fwd_kernel(f_ref, qp_ref, kp_ref, fq_ref, fk_ref):
    fq_ref[0] = jnp.dot(f_ref[...], qp_ref[0], precision=_HI)
    fk_ref[0] = jnp.dot(f_ref[...], kp_ref[0], precision=_HI)


def _forward_dft(fmat, qp3, kp3):
    bf, bc = 512, 512
    out = jax.ShapeDtypeStruct((B, L, D), jnp.float32)
    return pl.pallas_call(
        _fwd_kernel,
        grid=(B, L // bf, D // bc),
        in_specs=[
            pl.BlockSpec((bf, L), lambda b, i, j: (i, 0)),
            pl.BlockSpec((1, L, bc), lambda b, i, j: (b, 0, j)),
            pl.BlockSpec((1, L, bc), lambda b, i, j: (b, 0, j)),
        ],
        out_specs=[
            pl.BlockSpec((1, bf, bc), lambda b, i, j: (b, i, j)),
            pl.BlockSpec((1, bf, bc), lambda b, i, j: (b, i, j)),
        ],
        out_shape=[out, out],
    )(fmat, qp3, kp3)


def _point_kernel(fq_ref, fk_ref, p_ref):
    cq = fq_ref[0, :F, :]
    sq = fq_ref[0, F:, :]
    ck = fk_ref[0, :F, :]
    sk = fk_ref[0, F:, :]
    row0 = jax.lax.broadcasted_iota(jnp.int32, cq.shape, 0) == 0
    a = cq * ck + jnp.where(row0, 0.0, sq * sk)
    b = jnp.where(row0, sq * sk, cq * sk - sq * ck)
    p_ref[0, :F, :] = a
    p_ref[0, F:, :] = b


def _pointwise(fq, fk):
    bc = 512
    return pl.pallas_call(
        _point_kernel,
        grid=(B, D // bc),
        in_specs=[
            pl.BlockSpec((1, L, bc), lambda b, j: (b, 0, j)),
            pl.BlockSpec((1, L, bc), lambda b, j: (b, 0, j)),
        ],
        out_specs=pl.BlockSpec((1, L, bc), lambda b, j: (b, 0, j)),
        out_shape=jax.ShapeDtypeStruct((B, L, D), jnp.float32),
    )(fq, fk)


def _inv_kernel(i_ref, p_ref, mx_ref):
    ti = pl.program_id(2)
    corr = jnp.dot(i_ref[...], p_ref[0], precision=_HI)

    @pl.when(ti == 0)
    def _():
        row0 = jax.lax.broadcasted_iota(jnp.int32, corr.shape, 0) == 0
        c = jnp.where(row0, -jnp.inf, corr)
        mx_ref[0] = jnp.broadcast_to(jnp.max(c, axis=0, keepdims=True),
                                     mx_ref.shape[1:])

    @pl.when(ti != 0)
    def _():
        m = jnp.max(corr, axis=0, keepdims=True)
        mx_ref[0] = jnp.maximum(mx_ref[0], jnp.broadcast_to(m, mx_ref.shape[1:]))


def _inv_max(icat, pcat):
    bt, bc = 512, 512
    return pl.pallas_call(
        _inv_kernel,
        grid=(B, D // bc, L // bt),
        in_specs=[
            pl.BlockSpec((bt, L), lambda b, j, i: (i, 0)),
            pl.BlockSpec((1, L, bc), lambda b, j, i: (b, 0, j)),
        ],
        out_specs=pl.BlockSpec((1, 8, bc), lambda b, j, i: (b, 0, j)),
        out_shape=jax.ShapeDtypeStruct((B, 8, D), jnp.float32),
    )(icat, pcat)


def _fmt_kernel(cm_ref, out_ref):
    out_ref[0] = cm_ref[0]


def _fmt(cm128):
    return pl.pallas_call(
        _fmt_kernel,
        grid=(B,),
        in_specs=[pl.BlockSpec((1, H, DK, 128), lambda b: (b, 0, 0, 0))],
        out_specs=pl.BlockSpec((1, H, DK, 128), lambda b: (b, 0, 0, 0)),
        out_shape=jax.ShapeDtypeStruct((B, H, DK, L - 1), jnp.float32),
    )(cm128)


def _shifts(colmax):
    cm = colmax[:, 0, :]
    cm128 = jnp.broadcast_to(cm.reshape(B, H, DK, 1), (B, H, DK, 128))
    vals = _fmt(cm128)
    delays = vals[:, :, :, :7]
    delays = jnp.moveaxis(delays, -1, 1)
    sm = jax.nn.softmax(delays, axis=-1)
    shifts = sm[:, 0, :, :].astype(jnp.int32)
    return shifts.astype(jnp.float32).reshape(B, 1, D)


def _out_kernel(sh_ref, vp_ref, vprev_ref, wo_ref, bo_ref, o_ref):
    sh = sh_ref[0]
    vt = vp_ref[0]
    prev_last = vprev_ref[0, -1:, :]
    rolled = jnp.concatenate([prev_last, vt[:-1, :]], axis=0)
    sel = jnp.where(sh > 0.5, rolled, vt)
    o_ref[0] = jnp.dot(sel.astype(jnp.bfloat16),
                       wo_ref[...].astype(jnp.bfloat16),
                       preferred_element_type=jnp.float32) + bo_ref[...]


def _final(shifts, vp3, wot, bo2):
    bm = 512
    nm = L // bm
    return pl.pallas_call(
        _out_kernel,
        grid=(B, nm),
        in_specs=[
            pl.BlockSpec((1, 1, D), lambda b, i: (b, 0, 0)),
            pl.BlockSpec((1, bm, D), lambda b, i: (b, i, 0)),
            pl.BlockSpec((1, bm, D), lambda b, i: (b, (i - 1) % nm, 0)),
            pl.BlockSpec((D, D), lambda b, i: (0, 0)),
            pl.BlockSpec((1, D), lambda b, i: (0, 0)),
        ],
        out_specs=pl.BlockSpec((1, bm, D), lambda b, i: (b, i, 0)),
        out_shape=jax.ShapeDtypeStruct((B, L, D), jnp.float32),
    )(shifts, vp3, vp3, wot, bo2)


def kernel(q, k, v, Wq, bq, Wk, bk, Wv, bv, Wo, bo):
    fmat, icat = _dft_matrices()
    q2 = q.reshape(B * L, D)
    k2 = k.reshape(B * L, D)
    v2 = v.reshape(B * L, D)
    qp, kp, vp = _projections(q2, k2, v2, Wq.T, Wk.T, Wv.T,
                              bq.reshape(1, D), bk.reshape(1, D),
                              bv.reshape(1, D))
    qp3 = qp.reshape(B, L, D)
    kp3 = kp.reshape(B, L, D)
    vp3 = vp.reshape(B, L, D)
    fq, fk = _forward_dft(fmat, qp3, kp3)
    pcat = _pointwise(fq, fk)
    colmax = _inv_max(icat, pcat)
    sh = _shifts(colmax)
    out = _final(sh, vp3, Wo.T, bo.reshape(1, D))
    return out
